# baseline (speedup 1.0000x reference)
import jax
import jax.numpy as jnp
from jax import lax
from jax.experimental import pallas as pl
from jax.experimental.pallas import tpu as pltpu

_B, _N, _C = 2, 5000, 80
_R = _B * _C
_MAXD = 300
_IOU_THR = 0.5
_SCORE_THR = 0.01
_NP = 5120
_NEG_INF = float("-inf")


def _filter_kernel(boxes_ref, bsplit_ref, scores_ref, bo_ref, so_ref, lo_ref,
                   ms_ref, cs_ref, cx1_ref, cy1_ref, cx2_ref, cy2_ref):
    scores = scores_ref[...]
    ms_ref[...] = jnp.where(scores > _SCORE_THR, scores, _NEG_INF)

    xr = [[boxes_ref[4 * b + j: 4 * b + j + 1, :] for j in range(4)]
          for b in range(_B)]
    a2 = [jnp.maximum(xr[b][2] - xr[b][0], 0.0)
          * jnp.maximum(xr[b][3] - xr[b][1], 0.0) for b in range(_B)]
    idx = lax.broadcasted_iota(jnp.int32, (_R, _NP), 1)
    cidx = lax.broadcasted_iota(jnp.int32, (_R, _MAXD), 1)

    bsplit = bsplit_ref[...]

    def nms_step(t, carry):
        ms = ms_ref[...]
        m = jnp.max(ms, axis=1, keepdims=True)
        bi = jnp.min(jnp.where(ms == m, idx, _NP), axis=1,
                     keepdims=True)
        oh = (idx == bi) & (m > _NEG_INF)

        ohb16 = oh.astype(jnp.bfloat16)
        g = lax.dot_general(ohb16, bsplit, (((1,), (0,)), ((), ())),
                            preferred_element_type=jnp.float32)
        csum = (g[:, 0:8] + g[:, 8:16]) + g[:, 16:24]

        coords = []
        for b in range(_B):
            s = slice(_C * b, _C * (b + 1))
            x1r, y1r, x2r, y2r = xr[b]
            ohb = oh[s]
            bx1 = jnp.max(jnp.where(ohb, x1r, _NEG_INF), axis=1,
                          keepdims=True)
            by1 = jnp.max(jnp.where(ohb, y1r, _NEG_INF), axis=1,
                          keepdims=True)
            bx2 = jnp.max(jnp.where(ohb, x2r, _NEG_INF), axis=1,
                          keepdims=True)
            by2 = jnp.max(jnp.where(ohb, y2r, _NEG_INF), axis=1,
                          keepdims=True)
            coords.append((bx1, by1, bx2, by2))
            ix1 = jnp.maximum(bx1, x1r)
            iy1 = jnp.maximum(by1, y1r)
            ix2 = jnp.minimum(bx2, x2r)
            iy2 = jnp.minimum(by2, y2r)
            inter = jnp.maximum(ix2 - ix1, 0.0) * jnp.maximum(iy2 - iy1, 0.0)
            a1 = jnp.maximum(bx2 - bx1, 0.0) * jnp.maximum(by2 - by1, 0.0)
            union = a1 + a2[b] - inter
            suppress = inter / union > _IOU_THR
            ms_ref[s, :] = jnp.where(suppress, _NEG_INF, ms[s])

        colmask = cidx == t
        cs_ref[...] = jnp.where(colmask, m, cs_ref[...])
        bx1 = jnp.concatenate([coords[0][0], coords[1][0]], axis=0)
        by1 = jnp.concatenate([coords[0][1], coords[1][1]], axis=0)
        bx2 = jnp.concatenate([coords[0][2], coords[1][2]], axis=0)
        by2 = jnp.concatenate([coords[0][3], coords[1][3]], axis=0)
        cx1_ref[...] = jnp.where(colmask, bx1, cx1_ref[...])
        cy1_ref[...] = jnp.where(colmask, by1, cy1_ref[...])
        cx2_ref[...] = jnp.where(colmask, bx2, cx2_ref[...])
        cy2_ref[...] = jnp.where(colmask, by2, cy2_ref[...])
        return carry

    lax.fori_loop(0, _MAXD, nms_step, 0)

    rows = lax.broadcasted_iota(jnp.int32, (_C, _MAXD), 0)
    fidx = rows * _MAXD + cidx[:_C]
    oidx1 = lax.broadcasted_iota(jnp.int32, (1, _MAXD), 1)
    oidx4 = lax.broadcasted_iota(jnp.int32, (4, _MAXD), 1)

    def merge_step(t, carry):
        omask1 = oidx1 == t
        omask4 = oidx4 == t
        for b in range(_B):
            s = slice(_C * b, _C * (b + 1))
            cs = cs_ref[s, :]
            m = jnp.max(cs, axis=(0, 1), keepdims=True)
            ok = m > _NEG_INF
            bi = jnp.min(jnp.where(cs == m, fidx, _C * _MAXD),
                         axis=(0, 1), keepdims=True)
            oh = fidx == bi
            cs_ref[s, :] = jnp.where(oh, _NEG_INF, cs)
            lab = jnp.sum(jnp.where(oh, rows, 0), axis=(0, 1), keepdims=True)
            wx1 = jnp.sum(jnp.where(oh, cx1_ref[s, :], 0.0), axis=(0, 1),
                          keepdims=True)
            wy1 = jnp.sum(jnp.where(oh, cy1_ref[s, :], 0.0), axis=(0, 1),
                          keepdims=True)
            wx2 = jnp.sum(jnp.where(oh, cx2_ref[s, :], 0.0), axis=(0, 1),
                          keepdims=True)
            wy2 = jnp.sum(jnp.where(oh, cy2_ref[s, :], 0.0), axis=(0, 1),
                          keepdims=True)
            so_ref[b] = jnp.where(omask1, jnp.where(ok, m, -1.0), so_ref[b])
            lo_ref[b] = jnp.where(omask1, jnp.where(ok, lab, -1), lo_ref[b])
            wcoord = jnp.concatenate([wx1, wy1, wx2, wy2], axis=0)
            bo_ref[b] = jnp.where(omask4, jnp.where(ok, wcoord, -1.0),
                                  bo_ref[b])
        return carry

    lax.fori_loop(0, _MAXD, merge_step, 0)


def kernel(boxes, classification):
    boxes_t = jnp.moveaxis(boxes, 2, 1).reshape(_B * 4, _N)
    scores_t = jnp.moveaxis(classification, 2, 1).reshape(_R, _N)
    pad = _NP - _N
    boxes_t = jnp.pad(boxes_t, ((0, 0), (0, pad)))
    scores_t = jnp.pad(scores_t, ((0, 0), (0, pad)))

    bT = boxes_t.T
    hi = bT.astype(jnp.bfloat16)
    r1 = bT - hi.astype(jnp.float32)
    mid = r1.astype(jnp.bfloat16)
    r2 = r1 - mid.astype(jnp.float32)
    lo_part = r2.astype(jnp.bfloat16)
    bsplit = jnp.concatenate([hi, mid, lo_part], axis=1)

    bo, so, lo = pl.pallas_call(
        _filter_kernel,
        out_shape=[
            jax.ShapeDtypeStruct((_B, 4, _MAXD), jnp.float32),
            jax.ShapeDtypeStruct((_B, 1, _MAXD), jnp.float32),
            jax.ShapeDtypeStruct((_B, 1, _MAXD), jnp.int32),
        ],
        scratch_shapes=[
            pltpu.VMEM((_R, _NP), jnp.float32),
            pltpu.VMEM((_R, _MAXD), jnp.float32),
            pltpu.VMEM((_R, _MAXD), jnp.float32),
            pltpu.VMEM((_R, _MAXD), jnp.float32),
            pltpu.VMEM((_R, _MAXD), jnp.float32),
            pltpu.VMEM((_R, _MAXD), jnp.float32),
        ],
    )(boxes_t, bsplit, scores_t)

    boxes_out = jnp.moveaxis(bo, 1, 2)
    scores_out = so[:, 0, :]
    labels_out = lo[:, 0, :]
    return boxes_out, scores_out, labels_out


# speedup vs baseline: 1.1452x; 1.1452x over previous
import jax
import jax.numpy as jnp
from jax import lax
from jax.experimental import pallas as pl
from jax.experimental.pallas import tpu as pltpu

_B, _N, _C = 2, 5000, 80
_R = _B * _C
_MAXD = 300
_IOU_THR = 0.5
_SCORE_THR = 0.01
_NP = 5120
_NEG_INF = float("-inf")


def _filter_kernel(boxes_ref, bsplit_ref, scores_ref, bo_ref, so_ref, lo_ref,
                   ms_ref, cs_ref, cx1_ref, cy1_ref, cx2_ref, cy2_ref):
    scores = scores_ref[...]
    ms_ref[...] = jnp.where(scores > _SCORE_THR, scores, _NEG_INF)

    xr = [[boxes_ref[4 * b + j: 4 * b + j + 1, :] for j in range(4)]
          for b in range(_B)]
    a2 = [jnp.maximum(xr[b][2] - xr[b][0], 0.0)
          * jnp.maximum(xr[b][3] - xr[b][1], 0.0) for b in range(_B)]
    idx = lax.broadcasted_iota(jnp.int32, (_R, _NP), 1)
    cidx = lax.broadcasted_iota(jnp.int32, (_R, _MAXD), 1)

    bsplit = [bsplit_ref[0], bsplit_ref[1]]

    def nms_step(t, carry):
        ms = ms_ref[...]
        m = jnp.max(ms, axis=1, keepdims=True)
        bi = jnp.min(jnp.where(ms == m, idx, _NP), axis=1,
                     keepdims=True)
        oh = (idx == bi) & (m > _NEG_INF)

        ohb16 = oh.astype(jnp.bfloat16)

        coords = []
        for b in range(_B):
            s = slice(_C * b, _C * (b + 1))
            x1r, y1r, x2r, y2r = xr[b]
            g = lax.dot_general(ohb16[s], bsplit[b],
                                (((1,), (0,)), ((), ())),
                                preferred_element_type=jnp.float32)
            csum = (g[:, 0:4] + g[:, 4:8]) + g[:, 8:12]
            bx1 = csum[:, 0:1]
            by1 = csum[:, 1:2]
            bx2 = csum[:, 2:3]
            by2 = csum[:, 3:4]
            coords.append((bx1, by1, bx2, by2))
            ix1 = jnp.maximum(bx1, x1r)
            iy1 = jnp.maximum(by1, y1r)
            ix2 = jnp.minimum(bx2, x2r)
            iy2 = jnp.minimum(by2, y2r)
            inter = jnp.maximum(ix2 - ix1, 0.0) * jnp.maximum(iy2 - iy1, 0.0)
            a1 = jnp.maximum(bx2 - bx1, 0.0) * jnp.maximum(by2 - by1, 0.0)
            union = a1 + a2[b] - inter
            suppress = inter / union > _IOU_THR
            ms_ref[s, :] = jnp.where(suppress, _NEG_INF, ms[s])

        colmask = cidx == t
        cs_ref[...] = jnp.where(colmask, m, cs_ref[...])
        bx1 = jnp.concatenate([coords[0][0], coords[1][0]], axis=0)
        by1 = jnp.concatenate([coords[0][1], coords[1][1]], axis=0)
        bx2 = jnp.concatenate([coords[0][2], coords[1][2]], axis=0)
        by2 = jnp.concatenate([coords[0][3], coords[1][3]], axis=0)
        cx1_ref[...] = jnp.where(colmask, bx1, cx1_ref[...])
        cy1_ref[...] = jnp.where(colmask, by1, cy1_ref[...])
        cx2_ref[...] = jnp.where(colmask, bx2, cx2_ref[...])
        cy2_ref[...] = jnp.where(colmask, by2, cy2_ref[...])
        return carry

    lax.fori_loop(0, _MAXD, nms_step, 0)

    rows = lax.broadcasted_iota(jnp.int32, (_C, _MAXD), 0)
    fidx = rows * _MAXD + cidx[:_C]
    oidx1 = lax.broadcasted_iota(jnp.int32, (1, _MAXD), 1)
    oidx4 = lax.broadcasted_iota(jnp.int32, (4, _MAXD), 1)

    def merge_step(t, carry):
        omask1 = oidx1 == t
        omask4 = oidx4 == t
        for b in range(_B):
            s = slice(_C * b, _C * (b + 1))
            cs = cs_ref[s, :]
            m = jnp.max(cs, axis=(0, 1), keepdims=True)
            ok = m > _NEG_INF
            bi = jnp.min(jnp.where(cs == m, fidx, _C * _MAXD),
                         axis=(0, 1), keepdims=True)
            oh = fidx == bi
            cs_ref[s, :] = jnp.where(oh, _NEG_INF, cs)
            lab = jnp.sum(jnp.where(oh, rows, 0), axis=(0, 1), keepdims=True)
            wx1 = jnp.sum(jnp.where(oh, cx1_ref[s, :], 0.0), axis=(0, 1),
                          keepdims=True)
            wy1 = jnp.sum(jnp.where(oh, cy1_ref[s, :], 0.0), axis=(0, 1),
                          keepdims=True)
            wx2 = jnp.sum(jnp.where(oh, cx2_ref[s, :], 0.0), axis=(0, 1),
                          keepdims=True)
            wy2 = jnp.sum(jnp.where(oh, cy2_ref[s, :], 0.0), axis=(0, 1),
                          keepdims=True)
            so_ref[b] = jnp.where(omask1, jnp.where(ok, m, -1.0), so_ref[b])
            lo_ref[b] = jnp.where(omask1, jnp.where(ok, lab, -1), lo_ref[b])
            wcoord = jnp.concatenate([wx1, wy1, wx2, wy2], axis=0)
            bo_ref[b] = jnp.where(omask4, jnp.where(ok, wcoord, -1.0),
                                  bo_ref[b])
        return carry

    lax.fori_loop(0, _MAXD, merge_step, 0)


def kernel(boxes, classification):
    boxes_t = jnp.moveaxis(boxes, 2, 1).reshape(_B * 4, _N)
    scores_t = jnp.moveaxis(classification, 2, 1).reshape(_R, _N)
    pad = _NP - _N
    boxes_t = jnp.pad(boxes_t, ((0, 0), (0, pad)))
    scores_t = jnp.pad(scores_t, ((0, 0), (0, pad)))

    bT = boxes_t.T
    hi = lax.optimization_barrier(bT.astype(jnp.bfloat16))
    r1 = bT - hi.astype(jnp.float32)
    mid = lax.optimization_barrier(r1.astype(jnp.bfloat16))
    r2 = r1 - mid.astype(jnp.float32)
    lo_part = lax.optimization_barrier(r2.astype(jnp.bfloat16))
    bsplit = jnp.stack(
        [jnp.concatenate([hi[:, 4 * b:4 * b + 4], mid[:, 4 * b:4 * b + 4],
                          lo_part[:, 4 * b:4 * b + 4]], axis=1)
         for b in range(_B)], axis=0)

    bo, so, lo = pl.pallas_call(
        _filter_kernel,
        out_shape=[
            jax.ShapeDtypeStruct((_B, 4, _MAXD), jnp.float32),
            jax.ShapeDtypeStruct((_B, 1, _MAXD), jnp.float32),
            jax.ShapeDtypeStruct((_B, 1, _MAXD), jnp.int32),
        ],
        scratch_shapes=[
            pltpu.VMEM((_R, _NP), jnp.float32),
            pltpu.VMEM((_R, _MAXD), jnp.float32),
            pltpu.VMEM((_R, _MAXD), jnp.float32),
            pltpu.VMEM((_R, _MAXD), jnp.float32),
            pltpu.VMEM((_R, _MAXD), jnp.float32),
            pltpu.VMEM((_R, _MAXD), jnp.float32),
        ],
    )(boxes_t, bsplit, scores_t)

    boxes_out = jnp.moveaxis(bo, 1, 2)
    scores_out = so[:, 0, :]
    labels_out = lo[:, 0, :]
    return boxes_out, scores_out, labels_out


# speedup vs baseline: 1.2129x; 1.0591x over previous
import jax
import jax.numpy as jnp
from jax import lax
from jax.experimental import pallas as pl
from jax.experimental.pallas import tpu as pltpu

_B, _N, _C = 2, 5000, 80
_R = _B * _C
_MAXD = 300
_IOU_THR = 0.5
_SCORE_THR = 0.01
_NP = 5120
_NEG_INF = float("-inf")


def _filter_kernel(boxes_ref, bsplit_ref, scores_ref, bo_ref, so_ref, lo_ref,
                   ms_ref, cs_ref, cx1_ref, cy1_ref, cx2_ref, cy2_ref):
    scores = scores_ref[...]
    ms_ref[...] = jnp.where(scores > _SCORE_THR, scores, _NEG_INF)

    xr = [[boxes_ref[4 * b + j: 4 * b + j + 1, :] for j in range(4)]
          for b in range(_B)]
    a2 = [jnp.maximum(xr[b][2] - xr[b][0], 0.0)
          * jnp.maximum(xr[b][3] - xr[b][1], 0.0) for b in range(_B)]
    idx = lax.broadcasted_iota(jnp.int32, (_R, _NP), 1)
    cidx = lax.broadcasted_iota(jnp.int32, (_R, _MAXD), 1)

    bsplit = [bsplit_ref[0], bsplit_ref[1]]

    def nms_step(t, carry):
        ms = ms_ref[...]
        m = jnp.max(ms, axis=1, keepdims=True)
        bi = jnp.min(jnp.where(ms == m, idx, _NP), axis=1,
                     keepdims=True)
        bi = jnp.where(m > _NEG_INF, bi, _NP)
        oh = idx == bi

        ohb16 = oh.astype(jnp.bfloat16)

        coords = []
        for b in range(_B):
            s = slice(_C * b, _C * (b + 1))
            x1r, y1r, x2r, y2r = xr[b]
            g = lax.dot_general(ohb16[s], bsplit[b],
                                (((1,), (0,)), ((), ())),
                                preferred_element_type=jnp.float32)
            csum = (g[:, 0:4] + g[:, 4:8]) + g[:, 8:12]
            bx1 = csum[:, 0:1]
            by1 = csum[:, 1:2]
            bx2 = csum[:, 2:3]
            by2 = csum[:, 3:4]
            coords.append((bx1, by1, bx2, by2))
            ix1 = jnp.maximum(bx1, x1r)
            iy1 = jnp.maximum(by1, y1r)
            ix2 = jnp.minimum(bx2, x2r)
            iy2 = jnp.minimum(by2, y2r)
            inter = jnp.maximum(ix2 - ix1, 0.0) * jnp.maximum(iy2 - iy1, 0.0)
            a1 = jnp.maximum(bx2 - bx1, 0.0) * jnp.maximum(by2 - by1, 0.0)
            union = a1 + a2[b] - inter
            suppress = inter / union > _IOU_THR
            ms_ref[s, :] = jnp.where(suppress, _NEG_INF, ms[s])

        colmask = cidx == t
        cs_ref[...] = jnp.where(colmask, m, cs_ref[...])
        bx1 = jnp.concatenate([coords[0][0], coords[1][0]], axis=0)
        by1 = jnp.concatenate([coords[0][1], coords[1][1]], axis=0)
        bx2 = jnp.concatenate([coords[0][2], coords[1][2]], axis=0)
        by2 = jnp.concatenate([coords[0][3], coords[1][3]], axis=0)
        cx1_ref[...] = jnp.where(colmask, bx1, cx1_ref[...])
        cy1_ref[...] = jnp.where(colmask, by1, cy1_ref[...])
        cx2_ref[...] = jnp.where(colmask, bx2, cx2_ref[...])
        cy2_ref[...] = jnp.where(colmask, by2, cy2_ref[...])
        return carry

    lax.fori_loop(0, _MAXD, nms_step, 0)

    rows = lax.broadcasted_iota(jnp.int32, (_C, _MAXD), 0)
    fidx = rows * _MAXD + cidx[:_C]
    oidx1 = lax.broadcasted_iota(jnp.int32, (1, _MAXD), 1)
    oidx4 = lax.broadcasted_iota(jnp.int32, (4, _MAXD), 1)

    def merge_step(t, carry):
        omask1 = oidx1 == t
        omask4 = oidx4 == t
        for b in range(_B):
            s = slice(_C * b, _C * (b + 1))
            cs = cs_ref[s, :]
            m = jnp.max(cs, axis=(0, 1), keepdims=True)
            ok = m > _NEG_INF
            bi = jnp.min(jnp.where(cs == m, fidx, _C * _MAXD),
                         axis=(0, 1), keepdims=True)
            oh = fidx == bi
            cs_ref[s, :] = jnp.where(oh, _NEG_INF, cs)
            lab = jnp.sum(jnp.where(oh, rows, 0), axis=(0, 1), keepdims=True)
            wx1 = jnp.sum(jnp.where(oh, cx1_ref[s, :], 0.0), axis=(0, 1),
                          keepdims=True)
            wy1 = jnp.sum(jnp.where(oh, cy1_ref[s, :], 0.0), axis=(0, 1),
                          keepdims=True)
            wx2 = jnp.sum(jnp.where(oh, cx2_ref[s, :], 0.0), axis=(0, 1),
                          keepdims=True)
            wy2 = jnp.sum(jnp.where(oh, cy2_ref[s, :], 0.0), axis=(0, 1),
                          keepdims=True)
            so_ref[b] = jnp.where(omask1, jnp.where(ok, m, -1.0), so_ref[b])
            lo_ref[b] = jnp.where(omask1, jnp.where(ok, lab, -1), lo_ref[b])
            wcoord = jnp.concatenate([wx1, wy1, wx2, wy2], axis=0)
            bo_ref[b] = jnp.where(omask4, jnp.where(ok, wcoord, -1.0),
                                  bo_ref[b])
        return carry

    lax.fori_loop(0, _MAXD, merge_step, 0)


def kernel(boxes, classification):
    boxes_t = jnp.moveaxis(boxes, 2, 1).reshape(_B * 4, _N)
    scores_t = jnp.moveaxis(classification, 2, 1).reshape(_R, _N)
    pad = _NP - _N
    boxes_t = jnp.pad(boxes_t, ((0, 0), (0, pad)))
    scores_t = jnp.pad(scores_t, ((0, 0), (0, pad)))

    bT = boxes_t.T
    hi = lax.optimization_barrier(bT.astype(jnp.bfloat16))
    r1 = bT - hi.astype(jnp.float32)
    mid = lax.optimization_barrier(r1.astype(jnp.bfloat16))
    r2 = r1 - mid.astype(jnp.float32)
    lo_part = lax.optimization_barrier(r2.astype(jnp.bfloat16))
    bsplit = jnp.stack(
        [jnp.concatenate([hi[:, 4 * b:4 * b + 4], mid[:, 4 * b:4 * b + 4],
                          lo_part[:, 4 * b:4 * b + 4]], axis=1)
         for b in range(_B)], axis=0)

    bo, so, lo = pl.pallas_call(
        _filter_kernel,
        out_shape=[
            jax.ShapeDtypeStruct((_B, 4, _MAXD), jnp.float32),
            jax.ShapeDtypeStruct((_B, 1, _MAXD), jnp.float32),
            jax.ShapeDtypeStruct((_B, 1, _MAXD), jnp.int32),
        ],
        scratch_shapes=[
            pltpu.VMEM((_R, _NP), jnp.float32),
            pltpu.VMEM((_R, _MAXD), jnp.float32),
            pltpu.VMEM((_R, _MAXD), jnp.float32),
            pltpu.VMEM((_R, _MAXD), jnp.float32),
            pltpu.VMEM((_R, _MAXD), jnp.float32),
            pltpu.VMEM((_R, _MAXD), jnp.float32),
        ],
    )(boxes_t, bsplit, scores_t)

    boxes_out = jnp.moveaxis(bo, 1, 2)
    scores_out = so[:, 0, :]
    labels_out = lo[:, 0, :]
    return boxes_out, scores_out, labels_out
